# baseline (speedup 1.0000x reference)
import jax
import jax.numpy as jnp
from jax import lax
from jax.experimental import pallas as pl
from jax.experimental.pallas import tpu as pltpu
from jax.experimental.pallas import tpu_sc as plsc

VOCAB = 1000000
WORD_DIM = 128
BATCH = 4096
SEQ = 50

NC = 2
NS = 16
NW = NC * NS

B = BATCH * SEQ
B_PER_W = B // NW
CHUNK = 128
N_CHUNKS = B_PER_W // CHUNK
NBUF = 4


def _gather_body(x_hbm, table_hbm, out_hbm,
                 idx_v, buf0, buf1, buf2, buf3, g0, g1, g2, g3):
    bufs = (buf0, buf1, buf2, buf3)
    gsem = (g0, g1, g2, g3)
    wid = lax.axis_index("s") * NC + lax.axis_index("c")
    base = wid * B_PER_W
    pltpu.sync_copy(x_hbm.at[wid], idx_v)

    for b in range(NBUF - 1):
        pltpu.async_copy(table_hbm.at[idx_v.at[b]], bufs[b], gsem[b])

    def step(j, _):
        for p in range(NBUF):
            @pl.when(j % NBUF == p)
            def _(p=p):
                q = (p + NBUF - 1) % NBUF
                @pl.when(j + NBUF - 1 < N_CHUNKS)
                def _():
                    pltpu.async_copy(
                        table_hbm.at[idx_v.at[j + NBUF - 1]], bufs[q],
                        gsem[q])
                pltpu.make_async_copy(
                    table_hbm.at[idx_v.at[j]], bufs[p], gsem[p]).wait()
                pltpu.sync_copy(
                    bufs[p], out_hbm.at[pl.ds(base + j * CHUNK, CHUNK)])

        return 0

    lax.fori_loop(0, N_CHUNKS, step, 0)


@jax.jit
def _embed(x_flat, table):
    mesh = plsc.VectorSubcoreMesh(core_axis_name="c", subcore_axis_name="s")
    run = pl.kernel(
        _gather_body,
        out_type=jax.ShapeDtypeStruct((B, WORD_DIM), jnp.float32),
        mesh=mesh,
        scratch_types=[
            pltpu.VMEM((N_CHUNKS, CHUNK), jnp.int32),
            pltpu.VMEM((CHUNK, WORD_DIM), jnp.float32),
            pltpu.VMEM((CHUNK, WORD_DIM), jnp.float32),
            pltpu.VMEM((CHUNK, WORD_DIM), jnp.float32),
            pltpu.VMEM((CHUNK, WORD_DIM), jnp.float32),
            pltpu.SemaphoreType.DMA,
            pltpu.SemaphoreType.DMA,
            pltpu.SemaphoreType.DMA,
            pltpu.SemaphoreType.DMA,
        ],
    )
    return run(x_flat, table)


def kernel(x, lengths, table):
    x_flat = x.reshape(NW, N_CHUNKS, CHUNK)
    out = _embed(x_flat, table)
    emb = out.reshape(BATCH, SEQ, WORD_DIM)
    return (emb, lengths, emb)
